# baseline (speedup 1.0000x reference)
import functools

import jax
import jax.numpy as jnp
from jax import lax
from jax.experimental import pallas as pl
from jax.experimental.pallas import tpu as pltpu
from jax.experimental.pallas import tpu_sc as plsc

NUM_CORES = 2
NUM_SUBCORES = 16
NUM_WORKERS = NUM_CORES * NUM_SUBCORES

CHUNK = 128
NBUF = 4


@functools.partial(jax.jit, static_argnames=("tot", "dim"))
def _sc_gather(table, idx, tot, dim):
    per_w = tot // NUM_WORKERS
    nch = per_w // CHUNK

    def body(table_hbm, idx_hbm, out_hbm, idx_v, rows, gsem):
        c = lax.axis_index("c")
        s = lax.axis_index("s")
        wid = s * NUM_CORES + c
        base = wid * per_w

        pltpu.sync_copy(idx_hbm.at[pl.ds(base, per_w)], idx_v)

        def start_gather(g, b):
            off = pl.multiple_of(g * CHUNK, 8)
            pltpu.make_async_copy(
                table_hbm.at[idx_v.at[pl.ds(off, CHUNK)]],
                rows.at[b],
                gsem.at[b],
            ).start()

        def wait_gather(g, b):
            off = pl.multiple_of(g * CHUNK, 8)
            pltpu.make_async_copy(
                table_hbm.at[idx_v.at[pl.ds(off, CHUNK)]],
                rows.at[b],
                gsem.at[b],
            ).wait()

        for b in range(NBUF):
            start_gather(b, b)

        def step(i, carry):
            g0 = i * NBUF
            for b in range(NBUF):
                g = g0 + b
                wait_gather(g, b)
                pltpu.sync_copy(
                    rows.at[b], out_hbm.at[pl.ds(base + g * CHUNK, CHUNK)]
                )
                nxt = g + NBUF

                @pl.when(nxt < nch)
                def _():
                    start_gather(nxt, b)

            return carry

        lax.fori_loop(0, nch // NBUF, step, 0)

    run = pl.kernel(
        body,
        out_type=jax.ShapeDtypeStruct((tot, dim), jnp.float32),
        mesh=plsc.VectorSubcoreMesh(core_axis_name="c", subcore_axis_name="s"),
        scratch_types=[
            pltpu.VMEM((per_w,), jnp.int32),
            pltpu.VMEM((NBUF, CHUNK, dim), jnp.float32),
            pltpu.SemaphoreType.DMA((NBUF,)),
        ],
        compiler_params=pltpu.CompilerParams(use_tc_tiling_on_sc=False),
    )
    return run(table, idx)


def kernel(indices, table):
    tot = indices.size
    dim = table.shape[1]
    idx = indices.reshape(-1).astype(jnp.int32)
    out = _sc_gather(table, idx, tot, dim)
    return out.reshape(indices.shape + (dim,))


# speedup vs baseline: 1.0015x; 1.0015x over previous
import functools

import jax
import jax.numpy as jnp
from jax import lax
from jax.experimental import pallas as pl
from jax.experimental.pallas import tpu as pltpu
from jax.experimental.pallas import tpu_sc as plsc

NUM_CORES = 2
NUM_SUBCORES = 16
NUM_WORKERS = NUM_CORES * NUM_SUBCORES

CHUNK = 128
NBUF = 8
LOOKAHEAD = 6


@functools.partial(jax.jit, static_argnames=("tot", "dim"))
def _sc_gather(table, idx, tot, dim):
    per_w = tot // NUM_WORKERS
    nch = per_w // CHUNK

    def body(table_hbm, idx_hbm, out_hbm, idx_v, rows, gsem, osem):
        c = lax.axis_index("c")
        s = lax.axis_index("s")
        wid = s * NUM_CORES + c
        base = wid * per_w

        pltpu.sync_copy(idx_hbm.at[pl.ds(base, per_w)], idx_v)

        def gather(g, b):
            off = pl.multiple_of(g * CHUNK, 8)
            return pltpu.make_async_copy(
                table_hbm.at[idx_v.at[pl.ds(off, CHUNK)]],
                rows.at[b],
                gsem.at[b],
            )

        def out_copy(g, b):
            return pltpu.make_async_copy(
                rows.at[b],
                out_hbm.at[pl.ds(base + g * CHUNK, CHUNK)],
                osem.at[b],
            )

        for g in range(LOOKAHEAD):
            gather(g, g % NBUF).start()

        def step(i, carry):
            g0 = i * NBUF
            for b in range(NBUF):
                g = g0 + b
                nxt = g + LOOKAHEAD
                bn = (b + LOOKAHEAD) % NBUF

                @pl.when(nxt < nch)
                def _():
                    @pl.when(nxt >= NBUF)
                    def _():
                        out_copy(nxt - NBUF, bn).wait()

                    gather(nxt, bn).start()

                gather(g, b).wait()
                out_copy(g, b).start()

            return carry

        lax.fori_loop(0, nch // NBUF, step, 0)

        for g in range(nch - NBUF, nch):
            out_copy(g, g % NBUF).wait()

    run = pl.kernel(
        body,
        out_type=jax.ShapeDtypeStruct((tot, dim), jnp.float32),
        mesh=plsc.VectorSubcoreMesh(core_axis_name="c", subcore_axis_name="s"),
        scratch_types=[
            pltpu.VMEM((per_w,), jnp.int32),
            pltpu.VMEM((NBUF, CHUNK, dim), jnp.float32),
            pltpu.SemaphoreType.DMA((NBUF,)),
            pltpu.SemaphoreType.DMA((NBUF,)),
        ],
        compiler_params=pltpu.CompilerParams(use_tc_tiling_on_sc=False),
    )
    return run(table, idx)


def kernel(indices, table):
    tot = indices.size
    dim = table.shape[1]
    idx = indices.reshape(-1).astype(jnp.int32)
    out = _sc_gather(table, idx, tot, dim)
    return out.reshape(indices.shape + (dim,))


# speedup vs baseline: 1.0258x; 1.0243x over previous
import functools

import jax
import jax.numpy as jnp
from jax import lax
from jax.experimental import pallas as pl
from jax.experimental.pallas import tpu as pltpu
from jax.experimental.pallas import tpu_sc as plsc

NUM_CORES = 2
NUM_SUBCORES = 16
NUM_WORKERS = NUM_CORES * NUM_SUBCORES

CHUNK = 128
NBUF = 8
LOOKAHEAD = 6


@functools.partial(jax.jit, static_argnames=("tot", "dim"))
def _sc_gather(table, idx, tot, dim):
    per_w = tot // NUM_WORKERS
    nch = per_w // CHUNK

    def body(table_hbm, idx_hbm, out_hbm, idx_v, rows, gsem, osem):
        c = lax.axis_index("c")
        s = lax.axis_index("s")
        wid = s * NUM_CORES + c
        base = wid * per_w

        pltpu.sync_copy(idx_hbm.at[pl.ds(base, per_w)], idx_v)

        def gather(g, b):
            off = pl.multiple_of(g * CHUNK, 8)
            return pltpu.make_async_copy(
                table_hbm.at[idx_v.at[pl.ds(off, CHUNK)]],
                rows.at[b],
                gsem.at[b],
            )

        def out_copy(g, b):
            return pltpu.make_async_copy(
                rows.at[b],
                out_hbm.at[pl.ds(base + g * CHUNK, CHUNK)],
                osem.at[b],
            )

        for g in range(LOOKAHEAD):
            gather(g, g % NBUF).start()

        def step(i, carry):
            g0 = i * NBUF
            for b in range(NBUF):
                g = g0 + b
                nxt = g + LOOKAHEAD
                bn = (b + LOOKAHEAD) % NBUF

                @pl.when(nxt < nch)
                def _():
                    @pl.when(nxt >= NBUF)
                    def _():
                        out_copy(nxt - NBUF, bn).wait()

                    gather(nxt, bn).start()

                gather(g, b).wait()
                out_copy(g, b).start()

            return carry

        lax.fori_loop(0, nch // NBUF, step, 0)

        for g in range(nch - NBUF, nch):
            out_copy(g, g % NBUF).wait()

    run = pl.kernel(
        body,
        out_type=jax.ShapeDtypeStruct((tot, dim), jnp.float32),
        mesh=plsc.VectorSubcoreMesh(core_axis_name="c", subcore_axis_name="s"),
        scratch_types=[
            pltpu.VMEM((per_w,), jnp.int32),
            pltpu.VMEM((NBUF, CHUNK, dim), jnp.float32),
            pltpu.SemaphoreType.DMA((NBUF,)),
            pltpu.SemaphoreType.DMA((NBUF,)),
        ],
        compiler_params=pltpu.CompilerParams(use_tc_tiling_on_sc=False),
    )
    return run(table, idx)


def kernel(indices, table):
    tot = indices.size
    dim = table.shape[1]
    b, s = indices.shape
    idx = indices.T.reshape(-1).astype(jnp.int32)
    out = _sc_gather(table, idx, tot, dim)
    return out.reshape(s, b, dim).transpose(1, 0, 2)
